# baseline (speedup 1.0000x reference)
import functools

import jax
import jax.numpy as jnp
from jax import lax
from jax.experimental import pallas as pl
from jax.experimental.pallas import tpu as pltpu
from jax.experimental.pallas import tpu_sc as plsc

_VOCAB = 1000000
_DIM = 32
_PAD = 128
_L = 16
_W = 128


def _sc_hash_gather_add(ids_t, prev_t, prev2_t, bw_p, tw_p, n_seq, n_batch):
    mesh = plsc.VectorSubcoreMesh(core_axis_name="c", subcore_axis_name="s")
    n_blk = n_batch // _W

    @functools.partial(
        pl.kernel,
        out_type=jax.ShapeDtypeStruct(
            (n_seq, _DIM // 8, n_blk, 8, _W), jnp.float32
        ),
        mesh=mesh,
        compiler_params=pltpu.CompilerParams(
            use_tc_tiling_on_sc=False, needs_layout_passes=False
        ),
        scratch_types=[
            pltpu.VMEM((_W,), jnp.int32),
            pltpu.VMEM((_W,), jnp.int32),
            pltpu.VMEM((_W, _PAD), jnp.float32),
            pltpu.VMEM((_W, _PAD), jnp.float32),
            pltpu.SemaphoreType.DMA,
            pltpu.SemaphoreType.DMA,
        ],
    )
    def k(ids_hbm, prev_hbm, prev2_hbm, bw_hbm, tw_hbm, out_hbm,
          bi_idx, tri_idx, rows_bi, rows_tri, s1, s2):
        def body(ids_v, prev_v, prev2_v, out_v):
            @pl.loop(0, _W, step=_L)
            def _(c):
                a = ids_v[0, pl.ds(c, _L)]
                p = prev_v[0, pl.ds(c, _L)]
                p2 = prev2_v[0, pl.ds(c, _L)]
                s = p * 131 + a
                bi_idx[pl.ds(c, _L)] = s % _VOCAB
                tri_idx[pl.ds(c, _L)] = (p2 * 173 + s) % _VOCAB

            c1 = pltpu.async_copy(bw_hbm.at[bi_idx], rows_bi, s1)
            c2 = pltpu.async_copy(tw_hbm.at[tri_idx], rows_tri, s2)
            c1.wait()
            c2.wait()

            @pl.loop(0, _DIM // 8)
            def _(ct):
                @pl.loop(0, 8)
                def _(cs):
                    c16 = jnp.full((_L,), ct * 8 + cs, jnp.int32)

                    @pl.loop(0, _W, step=_L)
                    def _(k0):
                        i16 = lax.iota(jnp.int32, _L) + k0
                        v1 = plsc.load_gather(rows_bi, [i16, c16])
                        v2 = plsc.load_gather(rows_tri, [i16, c16])
                        out_v[0, ct, 0, cs, pl.ds(k0, _L)] = v1 + v2

        pltpu.emit_pipeline(
            body,
            grid=(n_seq, n_blk),
            in_specs=[
                pl.BlockSpec((1, _W), lambda s, b: (s, b)),
                pl.BlockSpec((1, _W), lambda s, b: (s, b)),
                pl.BlockSpec((1, _W), lambda s, b: (s, b)),
            ],
            out_specs=[
                pl.BlockSpec(
                    (1, _DIM // 8, 1, 8, _W), lambda s, b: (s, 0, b, 0, 0)
                )
            ],
            core_axis_name=("c", "s"),
            dimension_semantics=(pltpu.PARALLEL, pltpu.PARALLEL),
        )(ids_hbm, prev_hbm, prev2_hbm, out_hbm)

    return k(ids_t, prev_t, prev2_t, bw_p, tw_p)


def kernel(ids, bigram_weight, tri_weight):
    ids = ids.astype(jnp.int32)
    n, m = ids.shape
    ids_t = ids.T
    prev_t = jnp.zeros_like(ids_t).at[1:, :].set(ids_t[:-1, :])
    prev2_t = jnp.zeros_like(ids_t).at[2:, :].set(ids_t[:-2, :])
    bw_p = jnp.pad(bigram_weight, ((0, 0), (0, _PAD - _DIM)))
    tw_p = jnp.pad(tri_weight, ((0, 0), (0, _PAD - _DIM)))
    out5 = _sc_hash_gather_add(ids_t, prev_t, prev2_t, bw_p, tw_p, m, n)
    return out5.transpose(2, 4, 0, 1, 3).reshape(n, m, _DIM)


# speedup vs baseline: 1.0904x; 1.0904x over previous
import functools

import jax
import jax.numpy as jnp
from jax import lax
from jax.experimental import pallas as pl
from jax.experimental.pallas import tpu as pltpu
from jax.experimental.pallas import tpu_sc as plsc

_VOCAB = 1000000
_DIM = 32
_L = 16
_W = 128


def _sc_hash_gather_add(ids_t, prev_t, prev2_t, bw_p, tw_p, n_seq, n_batch):
    mesh = plsc.VectorSubcoreMesh(core_axis_name="c", subcore_axis_name="s")
    n_blk = n_batch // _W

    @functools.partial(
        pl.kernel,
        out_type=jax.ShapeDtypeStruct(
            (n_seq, _DIM // 8, n_blk, 8, _W), jnp.float32
        ),
        mesh=mesh,
        compiler_params=pltpu.CompilerParams(
            use_tc_tiling_on_sc=False, needs_layout_passes=False
        ),
        scratch_types=[
            pltpu.VMEM((_W,), jnp.int32),
            pltpu.VMEM((_W,), jnp.int32),
            pltpu.VMEM((_W, _DIM), jnp.float32),
            pltpu.VMEM((_W, _DIM), jnp.float32),
            pltpu.SemaphoreType.DMA,
            pltpu.SemaphoreType.DMA,
        ],
    )
    def k(ids_hbm, prev_hbm, prev2_hbm, bw_hbm, tw_hbm, out_hbm,
          bi_idx, tri_idx, rows_bi, rows_tri, s1, s2):
        def body(ids_v, prev_v, prev2_v, out_v):
            @pl.loop(0, _W, step=_L)
            def _(c):
                a = ids_v[0, pl.ds(c, _L)]
                p = prev_v[0, pl.ds(c, _L)]
                p2 = prev2_v[0, pl.ds(c, _L)]
                s = p * 131 + a
                bi_idx[pl.ds(c, _L)] = s % _VOCAB
                tri_idx[pl.ds(c, _L)] = (p2 * 173 + s) % _VOCAB

            c1 = pltpu.async_copy(bw_hbm.at[bi_idx], rows_bi, s1)
            c2 = pltpu.async_copy(tw_hbm.at[tri_idx], rows_tri, s2)
            c1.wait()
            c2.wait()

            @pl.loop(0, _DIM // 8)
            def _(ct):
                @pl.loop(0, 8)
                def _(cs):
                    c16 = jnp.full((_L,), ct * 8 + cs, jnp.int32)

                    @pl.loop(0, _W, step=_L)
                    def _(k0):
                        i16 = lax.iota(jnp.int32, _L) + k0
                        v1 = plsc.load_gather(rows_bi, [i16, c16])
                        v2 = plsc.load_gather(rows_tri, [i16, c16])
                        out_v[0, ct, 0, cs, pl.ds(k0, _L)] = v1 + v2

        pltpu.emit_pipeline(
            body,
            grid=(n_seq, n_blk),
            in_specs=[
                pl.BlockSpec((1, _W), lambda s, b: (s, b)),
                pl.BlockSpec((1, _W), lambda s, b: (s, b)),
                pl.BlockSpec((1, _W), lambda s, b: (s, b)),
            ],
            out_specs=[
                pl.BlockSpec(
                    (1, _DIM // 8, 1, 8, _W), lambda s, b: (s, 0, b, 0, 0)
                )
            ],
            core_axis_name=("c", "s"),
            dimension_semantics=(pltpu.PARALLEL, pltpu.PARALLEL),
        )(ids_hbm, prev_hbm, prev2_hbm, out_hbm)

    return k(ids_t, prev_t, prev2_t, bw_p, tw_p)


def kernel(ids, bigram_weight, tri_weight):
    ids = ids.astype(jnp.int32)
    n, m = ids.shape
    ids_t = ids.T
    prev_t = jnp.zeros_like(ids_t).at[1:, :].set(ids_t[:-1, :])
    prev2_t = jnp.zeros_like(ids_t).at[2:, :].set(ids_t[:-2, :])
    out5 = _sc_hash_gather_add(
        ids_t, prev_t, prev2_t, bigram_weight, tri_weight, m, n
    )
    return out5.transpose(2, 4, 0, 1, 3).reshape(n, m, _DIM)
